# baseline (speedup 1.0000x reference)
import functools

import numpy as np
import jax
import jax.numpy as jnp
from jax import lax
from jax.experimental import pallas as pl
from jax.experimental.pallas import tpu as pltpu
from jax.experimental.pallas import tpu_sc as plsc

C = 64
D_BINS, H_FEAT, W_FEAT = 80, 47, 156
GX, GY, GZ = 160, 160, 16
DHW = D_BINS * H_FEAT * W_FEAT
N_VOX = GX * GY * GZ
PC_RANGE = (0.0, -25.6, -3.0, 51.2, 25.6, 1.0)
NUM_BINS = 80
DEPTH_MIN = 2.0
DEPTH_MAX = 46.8
OOB = -2.0
BIN_SIZE = 2.0 * (DEPTH_MAX - DEPTH_MIN) / (NUM_BINS * (1 + NUM_BINS))

_PC_MIN = np.array(PC_RANGE[:3], dtype=np.float32)
_PC_MAX = np.array(PC_RANGE[3:], dtype=np.float32)
_VOX_SZ = (_PC_MAX - _PC_MIN) / np.array([GX, GY, GZ], dtype=np.float32)

TBLK = 2560
NROW = GY * GX // 128
NT = N_VOX // 128
NW = 32
PER_W = N_VOX // NW
NBLK = 128
NCHUNK = PER_W // NBLK


WPAD = 160
DROWS = H_FEAT * WPAD
NTAB = D_BINS * DROWS


def _transpose_body(in_ref, out_ref):
    for h in range(H_FEAT):
        out_ref[pl.ds(h * WPAD, W_FEAT), :] = in_ref[0, :, 0, h, :].T.astype(
            jnp.bfloat16)


def _relayout_features(feat5d):
    return pl.pallas_call(
        _transpose_body,
        grid=(D_BINS,),
        in_specs=[pl.BlockSpec((1, C, 1, H_FEAT, W_FEAT),
                               lambda d: (0, 0, d, 0, 0))],
        out_specs=pl.BlockSpec((DROWS, C), lambda d: (d, 0)),
        out_shape=jax.ShapeDtypeStruct((NTAB, C), jnp.bfloat16),
    )(feat5d)


def _bf(x):
    return x.astype(jnp.bfloat16).astype(jnp.float32)


def _grid_body(par_ref, idx_ref, w_ref):
    z = pl.program_id(0)
    l2c = [par_ref[i] for i in range(12)]
    c2i = [par_ref[12 + i] for i in range(12)]
    c2i23 = par_ref[24]
    nw = par_ref[25]
    nh = par_ref[26]

    shape = (NROW, 128)
    n = (lax.broadcasted_iota(jnp.int32, shape, 0) * 128
         + lax.broadcasted_iota(jnp.int32, shape, 1))
    nf = n.astype(jnp.float32)
    iy0 = jnp.floor((nf + 0.5) * (1.0 / GX))
    ixf = (nf - iy0 * GX) + 0.5
    iyf = iy0 + 0.5
    izf = z.astype(jnp.float32) + 0.5
    lx = _bf(ixf * _VOX_SZ[0] + _PC_MIN[0])
    ly = _bf(iyf * _VOX_SZ[1] + _PC_MIN[1])
    lz = _bf(izf * _VOX_SZ[2] + _PC_MIN[2])

    cam = [l2c[4 * i] * lx + l2c[4 * i + 1] * ly + l2c[4 * i + 2] * lz
           + l2c[4 * i + 3] for i in range(3)]
    cb = [_bf(c) for c in cam]
    p0 = c2i[0] * cb[0] + c2i[1] * cb[1] + c2i[2] * cb[2] + c2i[3]
    p1 = c2i[4] * cb[0] + c2i[5] * cb[1] + c2i[6] * cb[2] + c2i[7]
    p2 = c2i[8] * cb[0] + c2i[9] * cb[1] + c2i[10] * cb[2] + c2i[11]

    u = p0 / p2
    v = p1 / p2
    depth = p2 - c2i23
    arg = 1.0 + 8.0 * (depth - DEPTH_MIN) / BIN_SIZE
    dbin = jnp.where(arg >= 0.0,
                     -0.5 + 0.5 * jnp.sqrt(jnp.maximum(arg, 0.0)),
                     jnp.nan)

    gu = u / nw * 2.0 - 1.0
    gv = v / nh * 2.0 - 1.0
    gd = dbin / jnp.float32(NUM_BINS - 1) * 2.0 - 1.0
    gu = jnp.where(jnp.isfinite(gu), gu, OOB)
    gv = jnp.where(jnp.isfinite(gv), gv, OOB)
    gd = jnp.where(jnp.isfinite(gd), gd, OOB)

    gx = (gu + 1.0) * 0.5 * (W_FEAT - 1)
    gy = (gv + 1.0) * 0.5 * (H_FEAT - 1)
    gz = (gd + 1.0) * 0.5 * (D_BINS - 1)

    x0 = jnp.floor(gx)
    y0 = jnp.floor(gy)
    z0 = jnp.floor(gz)

    def axis_terms(g, g0, hi):
        res = []
        for d in (0.0, 1.0):
            gi = g0 + d
            w_ = 1.0 - jnp.abs(g - gi)
            valid = ((gi >= 0.0) & (gi <= hi)).astype(jnp.float32)
            cl = jnp.clip(gi, 0.0, hi).astype(jnp.int32)
            res.append((w_ * valid, cl))
        return res

    ax = axis_terms(gx, x0, W_FEAT - 1)
    ay = axis_terms(gy, y0, H_FEAT - 1)
    az = axis_terms(gz, z0, D_BINS - 1)

    k = 0
    for dz in (0, 1):
        wz_, zc = az[dz]
        for dy in (0, 1):
            wy_, yc = ay[dy]
            for dx in (0, 1):
                wx_, xc = ax[dx]
                w_ref[0, :, k, :] = (wx_ * wy_) * wz_
                idx_ref[0, :, k, :] = (zc * H_FEAT + yc) * WPAD + xc
                k += 1


def _make_grid(params):
    return pl.pallas_call(
        _grid_body,
        grid=(GZ,),
        in_specs=[pl.BlockSpec(memory_space=pltpu.SMEM)],
        out_specs=[
            pl.BlockSpec((1, NROW, 8, 128), lambda z: (z, 0, 0, 0)),
            pl.BlockSpec((1, NROW, 8, 128), lambda z: (z, 0, 0, 0)),
        ],
        out_shape=[
            jax.ShapeDtypeStruct((GZ, NROW, 8, 128), jnp.int32),
            jax.ShapeDtypeStruct((GZ, NROW, 8, 128), jnp.float32),
        ],
    )(params)


CW = C // 2


GROUP = 4
GBLK = GROUP * NBLK


def _sc_gather_body(idx_hbm, w_hbm, table_hbm, out_hbm,
                    idx_v, w_v, rows_v, outt_v, gsem, osem, isem):
    cid = lax.axis_index("c")
    sid = lax.axis_index("s")
    wid = sid * 2 + cid
    base_w = wid * PER_W

    lanes2 = lax.iota(jnp.int32, 16) * 2
    chmap = [(lanes2 + off) * GBLK for off in (0, 1, 32, 33)]

    def load_idx(ch, buf):
        pltpu.async_copy(idx_hbm.at[wid * NCHUNK + ch], idx_v.at[buf],
                         isem.at[buf])

    def load_w(ch, buf):
        pltpu.async_copy(w_hbm.at[wid * NCHUNK + ch], w_v.at[buf],
                         isem.at[buf])

    def drain_loads(buf):
        pltpu.make_async_copy(idx_hbm.at[0], idx_v.at[buf],
                              isem.at[buf]).wait()
        pltpu.make_async_copy(w_hbm.at[0], w_v.at[buf], isem.at[buf]).wait()

    def fire_gathers(buf):
        for kk in range(8):
            pltpu.async_copy(table_hbm.at[idx_v.at[buf, kk]],
                             rows_v.at[buf, kk], gsem.at[buf])

    def drain_gathers(buf):
        for kk in range(8):
            pltpu.make_async_copy(table_hbm.at[idx_v.at[buf, kk]],
                                  rows_v.at[buf, kk], gsem.at[buf]).wait()

    def fire_out(p):
        base = base_w + p * GBLK
        for c in range(C):
            pltpu.async_copy(outt_v.at[pl.ds(c * GBLK, GBLK)],
                             out_hbm.at[c, pl.ds(base, GBLK)], osem)

    def drain_out():
        for c in range(C):
            pltpu.make_async_copy(outt_v.at[pl.ds(c * GBLK, GBLK)],
                                  out_hbm.at[c, pl.ds(base_w, GBLK)],
                                  osem).wait()

    def compute(buf, q):
        obase = q * NBLK

        def grp_body(g, vcarry):
            vb = g * 16
            wvecs = [w_v[buf, kk, pl.ds(vb, 16)] for kk in range(8)]
            for j in range(16):
                ws = [wvecs[kk][j] for kk in range(8)]
                vi = vb + j
                acc = [None, None, None, None]
                for kk in range(8):
                    for h in (0, 1):
                        pb = rows_v[buf, kk, vi, pl.ds(32 * h, 32)]
                        wi = plsc.bitcast(pb, jnp.int32)
                        ea = plsc.bitcast(wi << 16, jnp.float32)
                        eb = plsc.bitcast(wi & jnp.int32(-65536), jnp.float32)
                        pa, pbb = ws[kk] * ea, ws[kk] * eb
                        if kk == 0:
                            acc[2 * h], acc[2 * h + 1] = pa, pbb
                        else:
                            acc[2 * h] = acc[2 * h] + pa
                            acc[2 * h + 1] = acc[2 * h + 1] + pbb
                for q4 in range(4):
                    plsc.store_scatter(outt_v, [chmap[q4] + (obase + vi)],
                                       acc[q4])
            return vcarry

        lax.fori_loop(0, NBLK // 16, grp_body, 0)

    load_idx(0, 0)
    load_w(0, 0)
    drain_loads(0)
    fire_gathers(0)
    load_idx(1, 1)
    load_w(1, 1)

    def quad_body(p, carry):
        for q in range(GROUP):
            ch = p * GROUP + q
            b = ch & 1

            @pl.when(ch + 1 < NCHUNK)
            def _():
                drain_loads(1 - b)
                fire_gathers(1 - b)

            drain_gathers(b)

            @pl.when(ch + 2 < NCHUNK)
            def _():
                load_idx(ch + 2, b)

            if q == 0:
                @pl.when(p > 0)
                def _():
                    drain_out()

            compute(b, q)

            @pl.when(ch + 2 < NCHUNK)
            def _():
                load_w(ch + 2, b)

        fire_out(p)
        return carry

    lax.fori_loop(0, NCHUNK // GROUP, quad_body, 0)
    drain_out()


def _sc_gather(idx8, w8, table):
    mesh = plsc.VectorSubcoreMesh(core_axis_name="c", subcore_axis_name="s")
    fn = functools.partial(
        pl.kernel,
        mesh=mesh,
        out_type=jax.ShapeDtypeStruct((C, N_VOX), jnp.float32),
        scratch_types=[
            pltpu.VMEM((2, 8, NBLK), jnp.int32),
            pltpu.VMEM((2, 8, NBLK), jnp.float32),
            pltpu.VMEM((2, 8, NBLK, C), jnp.bfloat16),
            pltpu.VMEM((C * GBLK,), jnp.float32),
            pltpu.SemaphoreType.DMA((2,)),
            pltpu.SemaphoreType.DMA,
            pltpu.SemaphoreType.DMA((2,)),
        ],
        compiler_params=pltpu.CompilerParams(needs_layout_passes=False,
                                             use_tc_tiling_on_sc=False),
    )(_sc_gather_body)
    return fn(idx8, w8, table)


def kernel(frustum_features, lidar_to_cam, cam_to_img, image_shape):
    table = _relayout_features(frustum_features)

    l2cb = lidar_to_cam[0, :3].astype(jnp.bfloat16).astype(jnp.float32)
    c2ib = cam_to_img[0].astype(jnp.bfloat16).astype(jnp.float32)
    img = jnp.max(image_shape, axis=0).astype(jnp.float32)
    params = jnp.concatenate([
        l2cb.reshape(12),
        c2ib.reshape(12),
        jnp.stack([cam_to_img[0, 2, 3], img[1] - 1.0, img[0] - 1.0,
                   jnp.float32(0.0)]),
    ])

    idx8, w8 = _make_grid(params)
    idx8 = idx8.reshape(NT, 8, 128)
    w8 = w8.reshape(NT, 8, 128)

    out = _sc_gather(idx8, w8, table)
    return out.reshape(1, C, GZ, GY, GX)


# speedup vs baseline: 1.0011x; 1.0011x over previous
import functools

import numpy as np
import jax
import jax.numpy as jnp
from jax import lax
from jax.experimental import pallas as pl
from jax.experimental.pallas import tpu as pltpu
from jax.experimental.pallas import tpu_sc as plsc

C = 64
D_BINS, H_FEAT, W_FEAT = 80, 47, 156
GX, GY, GZ = 160, 160, 16
DHW = D_BINS * H_FEAT * W_FEAT
N_VOX = GX * GY * GZ
PC_RANGE = (0.0, -25.6, -3.0, 51.2, 25.6, 1.0)
NUM_BINS = 80
DEPTH_MIN = 2.0
DEPTH_MAX = 46.8
OOB = -2.0
BIN_SIZE = 2.0 * (DEPTH_MAX - DEPTH_MIN) / (NUM_BINS * (1 + NUM_BINS))

_PC_MIN = np.array(PC_RANGE[:3], dtype=np.float32)
_PC_MAX = np.array(PC_RANGE[3:], dtype=np.float32)
_VOX_SZ = (_PC_MAX - _PC_MIN) / np.array([GX, GY, GZ], dtype=np.float32)

TBLK = 2560
NROW = GY * GX // 128
NT = N_VOX // 128
NW = 32
PER_W = N_VOX // NW
NBLK = 128
NCHUNK = PER_W // NBLK


WPAD = 160
DROWS = H_FEAT * WPAD
NTAB = D_BINS * DROWS


def _transpose_body(in_ref, out_ref):
    for h in range(H_FEAT):
        out_ref[pl.ds(h * WPAD, W_FEAT), :] = in_ref[0, :, 0, h, :].T.astype(
            jnp.bfloat16)


def _relayout_features(feat5d):
    return pl.pallas_call(
        _transpose_body,
        grid=(D_BINS,),
        in_specs=[pl.BlockSpec((1, C, 1, H_FEAT, W_FEAT),
                               lambda d: (0, 0, d, 0, 0))],
        out_specs=pl.BlockSpec((DROWS, C), lambda d: (d, 0)),
        out_shape=jax.ShapeDtypeStruct((NTAB, C), jnp.bfloat16),
    )(feat5d)


def _bf(x):
    return x.astype(jnp.bfloat16).astype(jnp.float32)


def _grid_body(par_ref, idx_ref, w_ref):
    z = pl.program_id(0)
    l2c = [par_ref[i] for i in range(12)]
    c2i = [par_ref[12 + i] for i in range(12)]
    c2i23 = par_ref[24]
    nw = par_ref[25]
    nh = par_ref[26]

    shape = (NROW, 128)
    n = (lax.broadcasted_iota(jnp.int32, shape, 0) * 128
         + lax.broadcasted_iota(jnp.int32, shape, 1))
    nf = n.astype(jnp.float32)
    iy0 = jnp.floor((nf + 0.5) * (1.0 / GX))
    ixf = (nf - iy0 * GX) + 0.5
    iyf = iy0 + 0.5
    izf = z.astype(jnp.float32) + 0.5
    lx = _bf(ixf * _VOX_SZ[0] + _PC_MIN[0])
    ly = _bf(iyf * _VOX_SZ[1] + _PC_MIN[1])
    lz = _bf(izf * _VOX_SZ[2] + _PC_MIN[2])

    cam = [l2c[4 * i] * lx + l2c[4 * i + 1] * ly + l2c[4 * i + 2] * lz
           + l2c[4 * i + 3] for i in range(3)]
    cb = [_bf(c) for c in cam]
    p0 = c2i[0] * cb[0] + c2i[1] * cb[1] + c2i[2] * cb[2] + c2i[3]
    p1 = c2i[4] * cb[0] + c2i[5] * cb[1] + c2i[6] * cb[2] + c2i[7]
    p2 = c2i[8] * cb[0] + c2i[9] * cb[1] + c2i[10] * cb[2] + c2i[11]

    u = p0 / p2
    v = p1 / p2
    depth = p2 - c2i23
    arg = 1.0 + 8.0 * (depth - DEPTH_MIN) / BIN_SIZE
    dbin = jnp.where(arg >= 0.0,
                     -0.5 + 0.5 * jnp.sqrt(jnp.maximum(arg, 0.0)),
                     jnp.nan)

    gu = u / nw * 2.0 - 1.0
    gv = v / nh * 2.0 - 1.0
    gd = dbin / jnp.float32(NUM_BINS - 1) * 2.0 - 1.0
    gu = jnp.where(jnp.isfinite(gu), gu, OOB)
    gv = jnp.where(jnp.isfinite(gv), gv, OOB)
    gd = jnp.where(jnp.isfinite(gd), gd, OOB)

    gx = (gu + 1.0) * 0.5 * (W_FEAT - 1)
    gy = (gv + 1.0) * 0.5 * (H_FEAT - 1)
    gz = (gd + 1.0) * 0.5 * (D_BINS - 1)

    x0 = jnp.floor(gx)
    y0 = jnp.floor(gy)
    z0 = jnp.floor(gz)

    def axis_terms(g, g0, hi):
        res = []
        for d in (0.0, 1.0):
            gi = g0 + d
            w_ = 1.0 - jnp.abs(g - gi)
            valid = ((gi >= 0.0) & (gi <= hi)).astype(jnp.float32)
            cl = jnp.clip(gi, 0.0, hi).astype(jnp.int32)
            res.append((w_ * valid, cl))
        return res

    ax = axis_terms(gx, x0, W_FEAT - 1)
    ay = axis_terms(gy, y0, H_FEAT - 1)
    az = axis_terms(gz, z0, D_BINS - 1)

    k = 0
    for dz in (0, 1):
        wz_, zc = az[dz]
        for dy in (0, 1):
            wy_, yc = ay[dy]
            for dx in (0, 1):
                wx_, xc = ax[dx]
                w_ref[0, :, k, :] = (wx_ * wy_) * wz_
                idx_ref[0, :, k, :] = (zc * H_FEAT + yc) * WPAD + xc
                k += 1


def _make_grid(params):
    return pl.pallas_call(
        _grid_body,
        grid=(GZ,),
        in_specs=[pl.BlockSpec(memory_space=pltpu.SMEM)],
        out_specs=[
            pl.BlockSpec((1, NROW, 8, 128), lambda z: (z, 0, 0, 0)),
            pl.BlockSpec((1, NROW, 8, 128), lambda z: (z, 0, 0, 0)),
        ],
        out_shape=[
            jax.ShapeDtypeStruct((GZ, NROW, 8, 128), jnp.int32),
            jax.ShapeDtypeStruct((GZ, NROW, 8, 128), jnp.float32),
        ],
    )(params)


CW = C // 2


GROUP = 4
GBLK = GROUP * NBLK


def _sc_gather_body(idx_hbm, w_hbm, table_hbm, out_hbm,
                    idx_v, w_v, rows_v, outt_v, gsem, osem, isem):
    cid = lax.axis_index("c")
    sid = lax.axis_index("s")
    wid = sid * 2 + cid
    base_w = wid * PER_W

    lanes2 = lax.iota(jnp.int32, 16) * 2
    chmap = [(lanes2 + off) * GBLK for off in (0, 1, 32, 33)]

    def load_idx(ch, buf):
        pltpu.async_copy(idx_hbm.at[wid * NCHUNK + ch], idx_v.at[buf],
                         isem.at[buf])

    def load_w(ch, buf):
        pltpu.async_copy(w_hbm.at[wid * NCHUNK + ch], w_v.at[buf],
                         isem.at[buf])

    def drain_loads(buf):
        pltpu.make_async_copy(idx_hbm.at[0], idx_v.at[buf],
                              isem.at[buf]).wait()
        pltpu.make_async_copy(w_hbm.at[0], w_v.at[buf], isem.at[buf]).wait()

    def fire_gathers(buf):
        for kk in range(8):
            pltpu.async_copy(table_hbm.at[idx_v.at[buf, kk]],
                             rows_v.at[buf, kk], gsem.at[buf])

    def drain_gathers(buf):
        for kk in range(8):
            pltpu.make_async_copy(table_hbm.at[idx_v.at[buf, kk]],
                                  rows_v.at[buf, kk], gsem.at[buf]).wait()

    def fire_out(p):
        base = base_w + p * GBLK
        for c in range(C):
            pltpu.async_copy(outt_v.at[pl.ds(c * GBLK, GBLK)],
                             out_hbm.at[c, pl.ds(base, GBLK)], osem)

    def drain_out():
        for c in range(C):
            pltpu.make_async_copy(outt_v.at[pl.ds(c * GBLK, GBLK)],
                                  out_hbm.at[c, pl.ds(base_w, GBLK)],
                                  osem).wait()

    def compute(buf, q):
        obase = q * NBLK

        def grp_body(g, vcarry):
            vb = g * 16
            wvecs = [w_v[buf, kk, pl.ds(vb, 16)] for kk in range(8)]
            for j in range(16):
                ws = [wvecs[kk][j] for kk in range(8)]
                vi = vb + j
                acc = [None, None, None, None]
                for h in (0, 1):
                    prods = []
                    for kk in range(8):
                        pb = rows_v[buf, kk, vi, pl.ds(32 * h, 32)]
                        wi = plsc.bitcast(pb, jnp.int32)
                        ea = plsc.bitcast(wi << 16, jnp.float32)
                        eb = plsc.bitcast(wi & jnp.int32(-65536), jnp.float32)
                        prods.append((ws[kk] * ea, ws[kk] * eb))
                    for s in (0, 1):
                        p_ = [t[s] for t in prods]
                        while len(p_) > 1:
                            p_ = [p_[i] + p_[i + 1]
                                  for i in range(0, len(p_), 2)]
                        acc[2 * h + s] = p_[0]
                for q4 in range(4):
                    plsc.store_scatter(outt_v, [chmap[q4] + (obase + vi)],
                                       acc[q4])
            return vcarry

        lax.fori_loop(0, NBLK // 16, grp_body, 0)

    load_idx(0, 0)
    load_w(0, 0)
    drain_loads(0)
    fire_gathers(0)
    load_idx(1, 1)
    load_w(1, 1)

    def quad_body(p, carry):
        for q in range(GROUP):
            ch = p * GROUP + q
            b = ch & 1

            @pl.when(ch + 1 < NCHUNK)
            def _():
                drain_loads(1 - b)
                fire_gathers(1 - b)

            drain_gathers(b)

            @pl.when(ch + 2 < NCHUNK)
            def _():
                load_idx(ch + 2, b)

            if q == 0:
                @pl.when(p > 0)
                def _():
                    drain_out()

            compute(b, q)

            @pl.when(ch + 2 < NCHUNK)
            def _():
                load_w(ch + 2, b)

        fire_out(p)
        return carry

    lax.fori_loop(0, NCHUNK // GROUP, quad_body, 0)
    drain_out()


def _sc_gather(idx8, w8, table):
    mesh = plsc.VectorSubcoreMesh(core_axis_name="c", subcore_axis_name="s")
    fn = functools.partial(
        pl.kernel,
        mesh=mesh,
        out_type=jax.ShapeDtypeStruct((C, N_VOX), jnp.float32),
        scratch_types=[
            pltpu.VMEM((2, 8, NBLK), jnp.int32),
            pltpu.VMEM((2, 8, NBLK), jnp.float32),
            pltpu.VMEM((2, 8, NBLK, C), jnp.bfloat16),
            pltpu.VMEM((C * GBLK,), jnp.float32),
            pltpu.SemaphoreType.DMA((2,)),
            pltpu.SemaphoreType.DMA,
            pltpu.SemaphoreType.DMA((2,)),
        ],
        compiler_params=pltpu.CompilerParams(needs_layout_passes=False,
                                             use_tc_tiling_on_sc=False),
    )(_sc_gather_body)
    return fn(idx8, w8, table)


def kernel(frustum_features, lidar_to_cam, cam_to_img, image_shape):
    table = _relayout_features(frustum_features)

    l2cb = lidar_to_cam[0, :3].astype(jnp.bfloat16).astype(jnp.float32)
    c2ib = cam_to_img[0].astype(jnp.bfloat16).astype(jnp.float32)
    img = jnp.max(image_shape, axis=0).astype(jnp.float32)
    params = jnp.concatenate([
        l2cb.reshape(12),
        c2ib.reshape(12),
        jnp.stack([cam_to_img[0, 2, 3], img[1] - 1.0, img[0] - 1.0,
                   jnp.float32(0.0)]),
    ])

    idx8, w8 = _make_grid(params)
    idx8 = idx8.reshape(NT, 8, 128)
    w8 = w8.reshape(NT, 8, 128)

    out = _sc_gather(idx8, w8, table)
    return out.reshape(1, C, GZ, GY, GX)


# speedup vs baseline: 1.1419x; 1.1406x over previous
import functools

import numpy as np
import jax
import jax.numpy as jnp
from jax import lax
from jax.experimental import pallas as pl
from jax.experimental.pallas import tpu as pltpu
from jax.experimental.pallas import tpu_sc as plsc

C = 64
D_BINS, H_FEAT, W_FEAT = 80, 47, 156
GX, GY, GZ = 160, 160, 16
DHW = D_BINS * H_FEAT * W_FEAT
N_VOX = GX * GY * GZ
PC_RANGE = (0.0, -25.6, -3.0, 51.2, 25.6, 1.0)
NUM_BINS = 80
DEPTH_MIN = 2.0
DEPTH_MAX = 46.8
OOB = -2.0
BIN_SIZE = 2.0 * (DEPTH_MAX - DEPTH_MIN) / (NUM_BINS * (1 + NUM_BINS))

_PC_MIN = np.array(PC_RANGE[:3], dtype=np.float32)
_PC_MAX = np.array(PC_RANGE[3:], dtype=np.float32)
_VOX_SZ = (_PC_MAX - _PC_MIN) / np.array([GX, GY, GZ], dtype=np.float32)

TBLK = 2560
NROW = GY * GX // 128
NT = N_VOX // 128
NW = 32
PER_W = N_VOX // NW
NBLK = 128
NCHUNK = PER_W // NBLK


HROWS = W_FEAT * D_BINS
NTAB = H_FEAT * HROWS


def _transpose_body(in_ref, out_ref):
    for w in range(W_FEAT):
        out_ref[pl.ds(w * D_BINS, D_BINS), :] = in_ref[0, 0, w, :, :].T.astype(
            jnp.bfloat16)


def _relayout_features(feat5d):
    return pl.pallas_call(
        _transpose_body,
        grid=(H_FEAT,),
        in_specs=[pl.BlockSpec((1, 1, W_FEAT, C, D_BINS),
                               lambda h: (0, h, 0, 0, 0))],
        out_specs=pl.BlockSpec((HROWS, C), lambda h: (h, 0)),
        out_shape=jax.ShapeDtypeStruct((NTAB, C), jnp.bfloat16),
    )(feat5d)


def _bf(x):
    return x.astype(jnp.bfloat16).astype(jnp.float32)


def _grid_body(par_ref, idx_ref, w_ref):
    z = pl.program_id(0)
    l2c = [par_ref[i] for i in range(12)]
    c2i = [par_ref[12 + i] for i in range(12)]
    c2i23 = par_ref[24]
    nw = par_ref[25]
    nh = par_ref[26]

    shape = (NROW, 128)
    n = (lax.broadcasted_iota(jnp.int32, shape, 0) * 128
         + lax.broadcasted_iota(jnp.int32, shape, 1))
    nf = n.astype(jnp.float32)
    iy0 = jnp.floor((nf + 0.5) * (1.0 / GX))
    ixf = (nf - iy0 * GX) + 0.5
    iyf = iy0 + 0.5
    izf = z.astype(jnp.float32) + 0.5
    lx = _bf(ixf * _VOX_SZ[0] + _PC_MIN[0])
    ly = _bf(iyf * _VOX_SZ[1] + _PC_MIN[1])
    lz = _bf(izf * _VOX_SZ[2] + _PC_MIN[2])

    cam = [l2c[4 * i] * lx + l2c[4 * i + 1] * ly + l2c[4 * i + 2] * lz
           + l2c[4 * i + 3] for i in range(3)]
    cb = [_bf(c) for c in cam]
    p0 = c2i[0] * cb[0] + c2i[1] * cb[1] + c2i[2] * cb[2] + c2i[3]
    p1 = c2i[4] * cb[0] + c2i[5] * cb[1] + c2i[6] * cb[2] + c2i[7]
    p2 = c2i[8] * cb[0] + c2i[9] * cb[1] + c2i[10] * cb[2] + c2i[11]

    u = p0 / p2
    v = p1 / p2
    depth = p2 - c2i23
    arg = 1.0 + 8.0 * (depth - DEPTH_MIN) / BIN_SIZE
    dbin = jnp.where(arg >= 0.0,
                     -0.5 + 0.5 * jnp.sqrt(jnp.maximum(arg, 0.0)),
                     jnp.nan)

    gu = u / nw * 2.0 - 1.0
    gv = v / nh * 2.0 - 1.0
    gd = dbin / jnp.float32(NUM_BINS - 1) * 2.0 - 1.0
    gu = jnp.where(jnp.isfinite(gu), gu, OOB)
    gv = jnp.where(jnp.isfinite(gv), gv, OOB)
    gd = jnp.where(jnp.isfinite(gd), gd, OOB)

    gx = (gu + 1.0) * 0.5 * (W_FEAT - 1)
    gy = (gv + 1.0) * 0.5 * (H_FEAT - 1)
    gz = (gd + 1.0) * 0.5 * (D_BINS - 1)

    x0 = jnp.floor(gx)
    y0 = jnp.floor(gy)
    z0 = jnp.floor(gz)

    def axis_terms(g, g0, hi):
        res = []
        for d in (0.0, 1.0):
            gi = g0 + d
            w_ = 1.0 - jnp.abs(g - gi)
            valid = ((gi >= 0.0) & (gi <= hi)).astype(jnp.float32)
            cl = jnp.clip(gi, 0.0, hi).astype(jnp.int32)
            res.append((w_ * valid, cl))
        return res

    ax = axis_terms(gx, x0, W_FEAT - 1)
    ay = axis_terms(gy, y0, H_FEAT - 1)
    az = axis_terms(gz, z0, D_BINS - 1)

    k = 0
    for dz in (0, 1):
        wz_, zc = az[dz]
        for dy in (0, 1):
            wy_, yc = ay[dy]
            for dx in (0, 1):
                wx_, xc = ax[dx]
                w_ref[0, :, k, :] = (wx_ * wy_) * wz_
                idx_ref[0, :, k, :] = (yc * W_FEAT + xc) * D_BINS + zc
                k += 1


def _make_grid(params):
    return pl.pallas_call(
        _grid_body,
        grid=(GZ,),
        in_specs=[pl.BlockSpec(memory_space=pltpu.SMEM)],
        out_specs=[
            pl.BlockSpec((1, NROW, 8, 128), lambda z: (z, 0, 0, 0)),
            pl.BlockSpec((1, NROW, 8, 128), lambda z: (z, 0, 0, 0)),
        ],
        out_shape=[
            jax.ShapeDtypeStruct((GZ, NROW, 8, 128), jnp.int32),
            jax.ShapeDtypeStruct((GZ, NROW, 8, 128), jnp.float32),
        ],
    )(params)


CW = C // 2


GROUP = 4
GBLK = GROUP * NBLK


def _sc_gather_body(idx_hbm, w_hbm, table_hbm, out_hbm,
                    idx_v, w_v, rows_v, outt_v, gsem, osem, isem):
    cid = lax.axis_index("c")
    sid = lax.axis_index("s")
    wid = sid * 2 + cid
    base_w = wid * PER_W

    lanes2 = lax.iota(jnp.int32, 16) * 2
    chmap = [(lanes2 + off) * GBLK for off in (0, 1, 32, 33)]

    def load_idx(ch, buf):
        pltpu.async_copy(idx_hbm.at[wid * NCHUNK + ch], idx_v.at[buf],
                         isem.at[buf])

    def load_w(ch, buf):
        pltpu.async_copy(w_hbm.at[wid * NCHUNK + ch], w_v.at[buf],
                         isem.at[buf])

    def drain_loads(buf):
        pltpu.make_async_copy(idx_hbm.at[0], idx_v.at[buf],
                              isem.at[buf]).wait()
        pltpu.make_async_copy(w_hbm.at[0], w_v.at[buf], isem.at[buf]).wait()

    def fire_gathers(buf):
        for kk in range(8):
            pltpu.async_copy(table_hbm.at[idx_v.at[buf, kk]],
                             rows_v.at[buf, kk], gsem.at[buf])

    def drain_gathers(buf):
        for kk in range(8):
            pltpu.make_async_copy(table_hbm.at[idx_v.at[buf, kk]],
                                  rows_v.at[buf, kk], gsem.at[buf]).wait()

    def fire_out(p):
        base = base_w + p * GBLK
        for c in range(C):
            pltpu.async_copy(outt_v.at[pl.ds(c * GBLK, GBLK)],
                             out_hbm.at[c, pl.ds(base, GBLK)], osem)

    def drain_out():
        for c in range(C):
            pltpu.make_async_copy(outt_v.at[pl.ds(c * GBLK, GBLK)],
                                  out_hbm.at[c, pl.ds(base_w, GBLK)],
                                  osem).wait()

    def compute(buf, q):
        obase = q * NBLK

        def grp_body(g, vcarry):
            vb = g * 16
            wvecs = [w_v[buf, kk, pl.ds(vb, 16)] for kk in range(8)]
            for j in range(16):
                ws = [wvecs[kk][j] for kk in range(8)]
                vi = vb + j
                acc = [None, None, None, None]
                for h in (0, 1):
                    prods = []
                    for kk in range(8):
                        pb = rows_v[buf, kk, vi, pl.ds(32 * h, 32)]
                        wi = plsc.bitcast(pb, jnp.int32)
                        ea = plsc.bitcast(wi << 16, jnp.float32)
                        eb = plsc.bitcast(wi & jnp.int32(-65536), jnp.float32)
                        prods.append((ws[kk] * ea, ws[kk] * eb))
                    for s in (0, 1):
                        p_ = [t[s] for t in prods]
                        while len(p_) > 1:
                            p_ = [p_[i] + p_[i + 1]
                                  for i in range(0, len(p_), 2)]
                        acc[2 * h + s] = p_[0]
                for q4 in range(4):
                    plsc.store_scatter(outt_v, [chmap[q4] + (obase + vi)],
                                       acc[q4])
            return vcarry

        lax.fori_loop(0, NBLK // 16, grp_body, 0)

    load_idx(0, 0)
    load_w(0, 0)
    drain_loads(0)
    fire_gathers(0)
    load_idx(1, 1)
    load_w(1, 1)

    def quad_body(p, carry):
        for q in range(GROUP):
            ch = p * GROUP + q
            b = ch & 1

            @pl.when(ch + 1 < NCHUNK)
            def _():
                drain_loads(1 - b)
                fire_gathers(1 - b)

            drain_gathers(b)

            @pl.when(ch + 2 < NCHUNK)
            def _():
                load_idx(ch + 2, b)

            if q == 0:
                @pl.when(p > 0)
                def _():
                    drain_out()

            compute(b, q)

            @pl.when(ch + 2 < NCHUNK)
            def _():
                load_w(ch + 2, b)

        fire_out(p)
        return carry

    lax.fori_loop(0, NCHUNK // GROUP, quad_body, 0)
    drain_out()


def _sc_gather(idx8, w8, table):
    mesh = plsc.VectorSubcoreMesh(core_axis_name="c", subcore_axis_name="s")
    fn = functools.partial(
        pl.kernel,
        mesh=mesh,
        out_type=jax.ShapeDtypeStruct((C, N_VOX), jnp.float32),
        scratch_types=[
            pltpu.VMEM((2, 8, NBLK), jnp.int32),
            pltpu.VMEM((2, 8, NBLK), jnp.float32),
            pltpu.VMEM((2, 8, NBLK, C), jnp.bfloat16),
            pltpu.VMEM((C * GBLK,), jnp.float32),
            pltpu.SemaphoreType.DMA((2,)),
            pltpu.SemaphoreType.DMA,
            pltpu.SemaphoreType.DMA((2,)),
        ],
        compiler_params=pltpu.CompilerParams(needs_layout_passes=False,
                                             use_tc_tiling_on_sc=False),
    )(_sc_gather_body)
    return fn(idx8, w8, table)


def kernel(frustum_features, lidar_to_cam, cam_to_img, image_shape):
    ff_t = jnp.transpose(frustum_features, (0, 3, 4, 1, 2))
    table = _relayout_features(ff_t)

    l2cb = lidar_to_cam[0, :3].astype(jnp.bfloat16).astype(jnp.float32)
    c2ib = cam_to_img[0].astype(jnp.bfloat16).astype(jnp.float32)
    img = jnp.max(image_shape, axis=0).astype(jnp.float32)
    params = jnp.concatenate([
        l2cb.reshape(12),
        c2ib.reshape(12),
        jnp.stack([cam_to_img[0, 2, 3], img[1] - 1.0, img[0] - 1.0,
                   jnp.float32(0.0)]),
    ])

    idx8, w8 = _make_grid(params)
    idx8 = idx8.reshape(NT, 8, 128)
    w8 = w8.reshape(NT, 8, 128)

    out = _sc_gather(idx8, w8, table)
    return out.reshape(1, C, GZ, GY, GX)
